# baseline (speedup 1.0000x reference)
import functools

import jax
import jax.numpy as jnp
from jax import lax
from jax.experimental import pallas as pl
from jax.experimental.pallas import tpu as pltpu
from jax.experimental.pallas import tpu_sc as plsc

VOCAB = 1000000
EMB = 64
B = 4096
L = 50

W = 128
N = B * L
NW = 32
R_PER_W = N // NW
GRANULE = 128
N_GROUP = R_PER_W // GRANULE
VPG = GRANULE // 16


def _make_gather():
    mesh = plsc.VectorSubcoreMesh(core_axis_name="c", subcore_axis_name="s")

    @functools.partial(
        pl.kernel,
        mesh=mesh,
        out_type=jax.ShapeDtypeStruct((N, W), jnp.float32),
        scratch_types=[
            pltpu.VMEM((R_PER_W,), jnp.int32),
            pltpu.VMEM((2 * GRANULE, W), jnp.float32),
            pltpu.SemaphoreType.DMA,
            pltpu.SemaphoreType.DMA,
            pltpu.SemaphoreType.DMA,
        ],
        compiler_params=pltpu.CompilerParams(use_tc_tiling_on_sc=True),
    )
    def gather_kernel(idx_hbm, table_hbm, out_hbm, idx_v, rows_v, gsem, ssa, ssb):
        wid = lax.axis_index("s") * 2 + lax.axis_index("c")
        rbase = wid * R_PER_W
        pltpu.sync_copy(idx_hbm.at[pl.ds(rbase, R_PER_W)], idx_v)

        def fire_gathers(g, set_):
            for j in range(VPG):
                vec = idx_v[pl.ds(g * GRANULE + j * 16, 16)]
                pltpu.async_copy(
                    table_hbm.at[vec],
                    rows_v.at[pl.ds(set_ * GRANULE + j * 16, 16)],
                    gsem,
                )

        def wait_gathers(set_):
            pltpu.make_async_copy(
                out_hbm.at[pl.ds(0, GRANULE)],
                rows_v.at[pl.ds(set_ * GRANULE, GRANULE)],
                gsem,
            ).wait()

        def fire_store(g, set_, ssem):
            pltpu.async_copy(
                rows_v.at[pl.ds(set_ * GRANULE, GRANULE)],
                out_hbm.at[pl.ds(rbase + g * GRANULE, GRANULE)],
                ssem,
            )

        def wait_store(g, set_, ssem):
            pltpu.make_async_copy(
                rows_v.at[pl.ds(set_ * GRANULE, GRANULE)],
                out_hbm.at[pl.ds(rbase + g * GRANULE, GRANULE)],
                ssem,
            ).wait()

        fire_gathers(0, 0)
        wait_gathers(0)
        fire_store(0, 0, ssa)
        fire_gathers(1, 1)

        def body(p, carry):
            ga = 2 * p + 1
            gb = 2 * p + 2
            wait_gathers(1)
            fire_store(ga, 1, ssb)
            wait_store(ga - 1, 0, ssa)
            fire_gathers(gb, 0)
            wait_gathers(0)
            fire_store(gb, 0, ssa)
            wait_store(ga, 1, ssb)
            fire_gathers(gb + 1, 1)
            return carry

        lax.fori_loop(0, (N_GROUP - 2) // 2, body, 0)

        g_last = N_GROUP - 1
        wait_gathers(1)
        fire_store(g_last, 1, ssb)
        wait_store(g_last - 1, 0, ssa)
        wait_store(g_last, 1, ssb)

    return gather_kernel


_gather = _make_gather()


def kernel(input_variable, table):
    t2 = table.reshape(VOCAB // 2, W)
    flat = input_variable.reshape(N).astype(jnp.int32)
    pair = _gather(flat >> 1, t2)
    lo = pair[:, :EMB]
    hi = pair[:, EMB:]
    sel = jnp.where((flat & 1)[:, None] == 1, hi, lo)
    return sel.reshape(B, L, EMB)
